# Initial kernel scaffold; baseline (speedup 1.0000x reference)
import jax
import jax.numpy as jnp
from jax.experimental import pallas as pl


def kernel(X, y, pro_embed, W_ih, W_hh, b_ih, b_hh, W_out, b_out):
    raise NotImplementedError("write your pallas kernel here")


# speedup vs baseline: 3.1575x; 3.1575x over previous
import jax
import jax.numpy as jnp
from jax.experimental import pallas as pl
from jax.experimental.pallas import tpu as pltpu

P = 10000
E = 128
H = 128
B, S = 64, 200
HALF = B // 2
M = S * HALF
E2_ROWS = 2 * P + 8
WA_ROWS = P + 8
GU = 16
CH = 256


def _body(e2_idx, w_idx, wih_t, whh_t, b2, e2_hbm, wa_hbm,
          out_ref, e2_tab, wa_tab, x_sc, hs_sc, wbuf_a, wbuf_b, sems):
    cp0 = pltpu.make_async_copy(e2_hbm, e2_tab, sems.at[0])
    cp1 = pltpu.make_async_copy(wa_hbm, wa_tab, sems.at[1])
    cp0.start()
    cp1.start()
    cp0.wait()
    cp1.wait()

    off = pl.program_id(0) * M

    def egather(k, _):
        base = k * GU
        for i in range(GU):
            mi = base + i
            idx = e2_idx[off + mi]
            cb = pl.multiple_of((idx >> 3) << 3, 8)
            chunk = e2_tab[pl.ds(cb, 8), :]
            x_sc[pl.ds(mi, 1), :] = pltpu.roll(chunk, -(idx & 7), axis=0)[0:1, :]
        return 0
    jax.lax.fori_loop(0, M // GU, egather, 0)

    def step(s, carry):
        h, c = carry
        rs = pl.multiple_of(s * HALF, HALF)
        x_s = x_sc[pl.ds(rs, HALF), :]
        g = (jnp.dot(x_s, wih_t[...], preferred_element_type=jnp.float32)
             + jnp.dot(h, whh_t[...], preferred_element_type=jnp.float32)
             + b2[...])
        gi = jax.nn.sigmoid(g[:, 0:H])
        gf = jax.nn.sigmoid(g[:, H:2 * H])
        gg = jnp.tanh(g[:, 2 * H:3 * H])
        go = jax.nn.sigmoid(g[:, 3 * H:4 * H])
        c = gf * c + gi * gg
        h = go * jnp.tanh(c)
        hs_sc[pl.ds(rs, HALF), :] = h
        return (h, c)

    h0 = jnp.zeros((HALF, H), jnp.float32)
    jax.lax.fori_loop(0, S, step, (h0, h0))

    for k in range(M // CH):
        cb0 = k * CH
        wbuf = wbuf_a if (k % 2 == 0) else wbuf_b

        def wgather(t, _, wbuf=wbuf, cb0=cb0):
            basej = t * GU
            for i in range(GU):
                j = basej + i
                wi = w_idx[off + cb0 + j]
                wb = pl.multiple_of((wi >> 3) << 3, 8)
                chunk = wa_tab[pl.ds(wb, 8), :]
                wbuf[pl.ds(j, 1), :] = pltpu.roll(chunk, -(wi & 7), axis=0)[0:1, :]
            return 0
        jax.lax.fori_loop(0, CH // GU, wgather, 0)

        hc = hs_sc[cb0:cb0 + CH, :]
        wv = wbuf[...]
        r = jnp.sum(hc * wv[:, 0:H], axis=1, keepdims=True) + wv[:, H:H + 1]
        out_ref[0, cb0:cb0 + CH, :] = jax.nn.sigmoid(r)


def kernel(X, y, pro_embed, W_ih, W_hh, b_ih, b_hh, W_out, b_out):
    f32 = jnp.float32
    X = X.astype(jnp.int32)
    y = y.astype(jnp.int32)

    z = jnp.zeros((P, E), f32)
    e2 = jnp.concatenate([
        jnp.concatenate([pro_embed, z], axis=1),
        jnp.concatenate([z, pro_embed], axis=1),
        jnp.zeros((E2_ROWS - 2 * P, 2 * E), f32),
    ], axis=0)

    wa = jnp.zeros((WA_ROWS, 2 * E), f32)
    wa = wa.at[:P, 0:H].set(W_out)
    wa = wa.at[:P, H].set(b_out)
    wa = wa.at[P, H].set(-1e30)

    Xr = X.reshape(2, HALF, S).transpose(0, 2, 1)
    yr = y.reshape(2, HALF, S).transpose(0, 2, 1)
    e2_idx = jnp.where(yr == -1, 2 * P, Xr + yr * P).reshape(2 * M)
    Xn = jnp.concatenate([X[:, 1:], jnp.zeros((B, 1), jnp.int32)], axis=1)
    Xnr = Xn.reshape(2, HALF, S).transpose(0, 2, 1)
    w_idx = jnp.where(Xnr == 0, P, Xnr - 1).reshape(2 * M)

    wih_t = W_ih.T
    whh_t = W_hh.T
    b2 = (b_ih + b_hh).reshape(1, 4 * H)

    out = pl.pallas_call(
        _body,
        grid=(2,),
        in_specs=[
            pl.BlockSpec(memory_space=pltpu.SMEM),
            pl.BlockSpec(memory_space=pltpu.SMEM),
            pl.BlockSpec((2 * E, 4 * H), lambda c: (0, 0)),
            pl.BlockSpec((H, 4 * H), lambda c: (0, 0)),
            pl.BlockSpec((1, 4 * H), lambda c: (0, 0)),
            pl.BlockSpec(memory_space=pl.ANY),
            pl.BlockSpec(memory_space=pl.ANY),
        ],
        out_specs=pl.BlockSpec((1, M, 1), lambda c: (c, 0, 0)),
        out_shape=jax.ShapeDtypeStruct((2, M, 1), f32),
        scratch_shapes=[
            pltpu.VMEM((E2_ROWS, 2 * E), f32),
            pltpu.VMEM((WA_ROWS, 2 * E), f32),
            pltpu.VMEM((M, 2 * E), f32),
            pltpu.VMEM((M, H), f32),
            pltpu.VMEM((CH, 2 * E), f32),
            pltpu.VMEM((CH, 2 * E), f32),
            pltpu.SemaphoreType.DMA((2,)),
        ],
        compiler_params=pltpu.CompilerParams(
            dimension_semantics=("parallel",),
            vmem_limit_bytes=56 * 1024 * 1024,
        ),
        name="dkt_pebg_fused",
    )(e2_idx, w_idx, wih_t, whh_t, b2, e2, wa)

    res = out.reshape(2, S, HALF)[:, :S - 1, :]
    return res.transpose(0, 2, 1).reshape(B, S - 1)


# speedup vs baseline: 4.9482x; 1.5671x over previous
import jax
import jax.numpy as jnp
from jax.experimental import pallas as pl
from jax.experimental.pallas import tpu as pltpu

P = 10000
E = 128
H = 128
B, S = 64, 200
M = S * B
E2_ROWS = 2 * P + 8
WA_ROWS = P + 8
EIDX_LEN = (S + 4) * B
GU = 16
CH = 256


def _body(e2_idx, w_idx, wih_t, whh_t, b2, e2_hbm, wa_hbm,
          out_ref, e2_tab, wa_tab, xbuf_a, xbuf_b, hs_sc, wbuf_a, wbuf_b,
          sems):
    cp0 = pltpu.make_async_copy(e2_hbm, e2_tab, sems.at[0])
    cp1 = pltpu.make_async_copy(wa_hbm, wa_tab, sems.at[1])
    cp0.start()
    cp1.start()
    cp0.wait()
    cp1.wait()

    def gather_x(dst, s):
        base = s * B
        for i in range(B):
            idx = e2_idx[base + i]
            cb = pl.multiple_of((idx >> 3) << 3, 8)
            chunk = e2_tab[pl.ds(cb, 8), :]
            dst[pl.ds(i, 1), :] = pltpu.roll(chunk, -(idx & 7), axis=0)[0:1, :]

    gather_x(xbuf_a, 0)
    gather_x(xbuf_b, 1)

    wih = wih_t[...]
    whh = whh_t[...]
    bias = b2[...]

    def lstm_step(s, xbuf, h, c):
        g = (jnp.dot(xbuf[...], wih, preferred_element_type=jnp.float32)
             + jnp.dot(h, whh, preferred_element_type=jnp.float32)
             + bias)
        gi = jax.nn.sigmoid(g[:, 0:H])
        gf = jax.nn.sigmoid(g[:, H:2 * H])
        gg = jnp.tanh(g[:, 2 * H:3 * H])
        go = jax.nn.sigmoid(g[:, 3 * H:4 * H])
        c = gf * c + gi * gg
        h = go * jnp.tanh(c)
        hs_sc[pl.ds(pl.multiple_of(s * B, B), B), :] = h
        gather_x(xbuf, s + 2)
        return h, c

    def step2(t, carry):
        h, c = carry
        s0 = t * 2
        h, c = lstm_step(s0, xbuf_a, h, c)
        h, c = lstm_step(s0 + 1, xbuf_b, h, c)
        return (h, c)

    h0 = jnp.zeros((B, H), jnp.float32)
    jax.lax.fori_loop(0, S // 2, step2, (h0, h0))

    for k in range(M // CH):
        cb0 = k * CH
        wbuf = wbuf_a if (k % 2 == 0) else wbuf_b

        def wgather(t, _, wbuf=wbuf, cb0=cb0):
            basej = t * GU
            for i in range(GU):
                j = basej + i
                wi = w_idx[cb0 + j]
                wb = pl.multiple_of((wi >> 3) << 3, 8)
                chunk = wa_tab[pl.ds(wb, 8), :]
                wbuf[pl.ds(j, 1), :] = pltpu.roll(chunk, -(wi & 7), axis=0)[0:1, :]
            return 0
        jax.lax.fori_loop(0, CH // GU, wgather, 0)

        hc = hs_sc[cb0:cb0 + CH, :]
        wv = wbuf[...]
        r = jnp.sum(hc * wv[:, 0:H], axis=1, keepdims=True) + wv[:, H:H + 1]
        out_ref[cb0:cb0 + CH, :] = jax.nn.sigmoid(r)


def kernel(X, y, pro_embed, W_ih, W_hh, b_ih, b_hh, W_out, b_out):
    f32 = jnp.float32
    X = X.astype(jnp.int32)
    y = y.astype(jnp.int32)

    z = jnp.zeros((P, E), f32)
    e2 = jnp.concatenate([
        jnp.concatenate([pro_embed, z], axis=1),
        jnp.concatenate([z, pro_embed], axis=1),
        jnp.zeros((E2_ROWS - 2 * P, 2 * E), f32),
    ], axis=0)

    wa = jnp.zeros((WA_ROWS, 2 * E), f32)
    wa = wa.at[:P, 0:H].set(W_out)
    wa = wa.at[:P, H].set(b_out)
    wa = wa.at[P, H].set(-1e30)

    Xt = X.T
    yt = y.T
    e2_idx = jnp.where(yt == -1, 2 * P, Xt + yt * P).reshape(M)
    e2_idx = jnp.concatenate(
        [e2_idx, jnp.zeros((EIDX_LEN - M,), jnp.int32)])
    Xn = jnp.concatenate([X[:, 1:], jnp.zeros((B, 1), jnp.int32)], axis=1)
    w_idx = jnp.where(Xn.T == 0, P, Xn.T - 1).reshape(M)

    wih_t = W_ih.T
    whh_t = W_hh.T
    b2 = (b_ih + b_hh).reshape(1, 4 * H)

    out = pl.pallas_call(
        _body,
        in_specs=[
            pl.BlockSpec(memory_space=pltpu.SMEM),
            pl.BlockSpec(memory_space=pltpu.SMEM),
            pl.BlockSpec(memory_space=pltpu.VMEM),
            pl.BlockSpec(memory_space=pltpu.VMEM),
            pl.BlockSpec(memory_space=pltpu.VMEM),
            pl.BlockSpec(memory_space=pl.ANY),
            pl.BlockSpec(memory_space=pl.ANY),
        ],
        out_specs=pl.BlockSpec(memory_space=pltpu.VMEM),
        out_shape=jax.ShapeDtypeStruct((M, 1), f32),
        scratch_shapes=[
            pltpu.VMEM((E2_ROWS, 2 * E), f32),
            pltpu.VMEM((WA_ROWS, 2 * E), f32),
            pltpu.VMEM((B, 2 * E), f32),
            pltpu.VMEM((B, 2 * E), f32),
            pltpu.VMEM((M, H), f32),
            pltpu.VMEM((CH, 2 * E), f32),
            pltpu.VMEM((CH, 2 * E), f32),
            pltpu.SemaphoreType.DMA((2,)),
        ],
        compiler_params=pltpu.CompilerParams(
            vmem_limit_bytes=56 * 1024 * 1024,
        ),
        name="dkt_pebg_fused",
    )(e2_idx, w_idx, wih_t, whh_t, b2, e2, wa)

    return out.reshape(S, B)[:S - 1].T
